# baseline (speedup 1.0000x reference)
import jax
import jax.numpy as jnp
from jax.experimental import pallas as pl
from jax.experimental.pallas import tpu as pltpu

_N_DMA = 8


def _copy_body(table_ref, out_ref, *sems):
    rows = out_ref.shape[0]
    chunk = rows // _N_DMA
    copies = [
        pltpu.make_async_copy(
            table_ref.at[pl.ds(i * chunk, chunk)],
            out_ref.at[pl.ds(i * chunk, chunk)],
            sems[i],
        )
        for i in range(_N_DMA)
    ]
    for c in copies:
        c.start()
    for c in copies:
        c.wait()


def kernel(x, table):
    seq_len = x.shape[-1]
    hidden = table.shape[1]
    return pl.pallas_call(
        _copy_body,
        in_specs=[pl.BlockSpec(memory_space=pl.ANY)],
        out_specs=pl.BlockSpec(memory_space=pl.ANY),
        out_shape=jax.ShapeDtypeStruct((seq_len, hidden), table.dtype),
        scratch_shapes=[pltpu.SemaphoreType.DMA] * _N_DMA,
    )(table)


# speedup vs baseline: 40.4597x; 40.4597x over previous
import jax
import jax.numpy as jnp
from jax.experimental import pallas as pl


_BLOCK_ROWS = 1024


def _copy_block(table_ref, out_ref):
    out_ref[...] = table_ref[...]


def kernel(x, table):
    seq_len = x.shape[-1]
    hidden = table.shape[1]
    num_blocks = seq_len // _BLOCK_ROWS
    return pl.pallas_call(
        _copy_block,
        grid=(num_blocks,),
        in_specs=[pl.BlockSpec((_BLOCK_ROWS, hidden), lambda i: (i, 0))],
        out_specs=pl.BlockSpec((_BLOCK_ROWS, hidden), lambda i: (i, 0)),
        out_shape=jax.ShapeDtypeStruct((seq_len, hidden), table.dtype),
    )(table)


# speedup vs baseline: 43.4267x; 1.0733x over previous
import jax
import jax.numpy as jnp
from jax.experimental import pallas as pl


_BLOCK_ROWS = 2048


def _copy_block(table_ref, out_ref):
    out_ref[...] = table_ref[...]


def kernel(x, table):
    seq_len = x.shape[-1]
    hidden = table.shape[1]
    num_blocks = seq_len // _BLOCK_ROWS
    return pl.pallas_call(
        _copy_block,
        grid=(num_blocks,),
        in_specs=[pl.BlockSpec((_BLOCK_ROWS, hidden), lambda i: (i, 0))],
        out_specs=pl.BlockSpec((_BLOCK_ROWS, hidden), lambda i: (i, 0)),
        out_shape=jax.ShapeDtypeStruct((seq_len, hidden), table.dtype),
    )(table)
